# baseline (speedup 1.0000x reference)
import functools

import jax
import jax.numpy as jnp
from jax.experimental import pallas as pl
from jax.experimental.pallas import tpu as pltpu
from jax.experimental.pallas import tpu_sc as plsc

_B = 16384
_NC = 2
_NW = _NC * 16
_CHUNK = _B // _NW
_L = 16
_NBLK = 4


def kernel(x, weight):
    n_flat = weight.shape[0] * weight.shape[1]
    wflat = weight.reshape(n_flat)

    mesh = plsc.VectorSubcoreMesh(core_axis_name="c", subcore_axis_name="s",
                                  num_cores=_NC)
    out_sds = jax.ShapeDtypeStruct((_B,), jnp.float32)

    @functools.partial(
        pl.kernel,
        out_type=(out_sds, out_sds),
        mesh=mesh,
        scratch_types=[
            pltpu.VMEM((_CHUNK,), jnp.int32),
            pltpu.VMEM((n_flat,), jnp.float32),
            pltpu.VMEM((2, _CHUNK), jnp.float32),
            pltpu.SemaphoreType.DMA,
        ],
        compiler_params=pltpu.CompilerParams(needs_layout_passes=False,
                                             skip_device_barrier=True),
    )
    def _sc_lookup(x_hbm, w_hbm, ox_hbm, oy_hbm, idx_v, tab_v, o_v, sem):
        wid = jax.lax.axis_index("s") * _NC + jax.lax.axis_index("c")
        base = wid * _CHUNK
        cp_idx = pltpu.async_copy(x_hbm.at[pl.ds(base, _CHUNK)], idx_v, sem)
        cp_tab = pltpu.async_copy(w_hbm, tab_v, sem)
        cp_tab.wait()
        cp_idx.wait()

        @plsc.parallel_loop(0, _CHUNK, step=_L, unroll=4)
        def _(i):
            idx2 = idx_v[pl.ds(i, _L)] * 2
            o_v[0, pl.ds(i, _L)] = plsc.load_gather(tab_v, [idx2])
            o_v[1, pl.ds(i, _L)] = plsc.load_gather(tab_v, [idx2 + 1])

        cp_ox = pltpu.async_copy(o_v.at[0], ox_hbm.at[pl.ds(base, _CHUNK)], sem)
        cp_oy = pltpu.async_copy(o_v.at[1], oy_hbm.at[pl.ds(base, _CHUNK)], sem)
        cp_ox.wait()
        cp_oy.wait()

    return _sc_lookup(x, wflat)


# speedup vs baseline: 1.0983x; 1.0983x over previous
import functools

import jax
import jax.numpy as jnp
from jax.experimental import pallas as pl
from jax.experimental.pallas import tpu as pltpu
from jax.experimental.pallas import tpu_sc as plsc

_B = 16384
_NC = 1
_NW = _NC * 16
_CHUNK = _B // _NW
_L = 16
_NBLK = 4


def kernel(x, weight):
    n_flat = weight.shape[0] * weight.shape[1]
    wflat = weight.reshape(n_flat)

    mesh = plsc.VectorSubcoreMesh(core_axis_name="c", subcore_axis_name="s",
                                  num_cores=_NC)
    out_sds = jax.ShapeDtypeStruct((_B,), jnp.float32)

    @functools.partial(
        pl.kernel,
        out_type=(out_sds, out_sds),
        mesh=mesh,
        scratch_types=[
            pltpu.VMEM((_CHUNK,), jnp.int32),
            pltpu.VMEM((n_flat,), jnp.float32),
            pltpu.VMEM((2, _CHUNK), jnp.float32),
            pltpu.SemaphoreType.DMA,
        ],
        compiler_params=pltpu.CompilerParams(needs_layout_passes=False,
                                             skip_device_barrier=True),
    )
    def _sc_lookup(x_hbm, w_hbm, ox_hbm, oy_hbm, idx_v, tab_v, o_v, sem):
        wid = jax.lax.axis_index("s") * _NC + jax.lax.axis_index("c")
        base = wid * _CHUNK
        cp_idx = pltpu.async_copy(x_hbm.at[pl.ds(base, _CHUNK)], idx_v, sem)
        cp_tab = pltpu.async_copy(w_hbm, tab_v, sem)
        cp_tab.wait()
        cp_idx.wait()

        @plsc.parallel_loop(0, _CHUNK, step=_L, unroll=4)
        def _(i):
            idx2 = idx_v[pl.ds(i, _L)] * 2
            o_v[0, pl.ds(i, _L)] = plsc.load_gather(tab_v, [idx2])
            o_v[1, pl.ds(i, _L)] = plsc.load_gather(tab_v, [idx2 + 1])

        cp_ox = pltpu.async_copy(o_v.at[0], ox_hbm.at[pl.ds(base, _CHUNK)], sem)
        cp_oy = pltpu.async_copy(o_v.at[1], oy_hbm.at[pl.ds(base, _CHUNK)], sem)
        cp_ox.wait()
        cp_oy.wait()

    return _sc_lookup(x, wflat)
